# baseline (speedup 1.0000x reference)
import functools

import jax
import jax.numpy as jnp
from jax.experimental import pallas as pl
from jax.experimental.pallas import tpu as pltpu

_NEG_SLOPE = 0.01
_LANE = 128
_VMEM = 64 * 1024 * 1024

_BF = jnp.bfloat16
_F32 = jnp.float32


def _lrelu(x):
    return jnp.where(x > 0, x, _NEG_SLOPE * x)


def _ceil_to(n, m):
    return ((n + m - 1) // m) * m


def _dot(a, b):
    return jnp.dot(a, b, preferred_element_type=jnp.float32)


def _branch_body(x_ref, a_ref, w1_ref, w2_ref, mix_ref, d1t_ref, d1b_ref,
                 att_ref, h1_ref, h2_ref, y1_ref, h2s_ref, *, nrows):
    a = a_ref[0]
    att = att_ref[0]

    t1 = _dot(x_ref[0], w1_ref[0]).astype(_BF)
    h1 = _lrelu(_dot(a, t1))
    h1_ref[0] = h1[:nrows, :h1_ref.shape[2]]
    h1s = (att * h1).astype(_BF)

    t2 = _dot(h1.astype(_BF), w2_ref[0]).astype(_BF)
    h2 = _lrelu(_dot(a, t2))
    h2_ref[0] = h2[:nrows, :h2_ref.shape[2]]
    h2s_ref[0] = (att * h2).astype(_BF)

    y1 = _dot(mix_ref[0], d1t_ref[...]) + _dot(h1s, d1b_ref[...])
    y1_ref[0] = y1.astype(_BF)


def _hop1_body(al_ref, ar_ref, y1_ref, d2t_ref, d2b_ref, h2s_ref,
               d1_ref, y2_ref, *, nrows):
    d1 = _lrelu(_dot(al_ref[0], y1_ref[0]) + _dot(ar_ref[0], y1_ref[1]))
    d1_ref[0] = d1[:nrows, :d1_ref.shape[2]]
    y2 = _dot(d1.astype(_BF), d2t_ref[...]) + _dot(h2s_ref[0], d2b_ref[...])
    y2_ref[0] = y2.astype(_BF)


def _hop2_body(al_ref, ar_ref, y2_ref, d2_ref, *, nrows):
    d2 = _lrelu(_dot(al_ref[0], y2_ref[0]) + _dot(ar_ref[0], y2_ref[1]))
    d2_ref[0] = d2[:nrows, :d2_ref.shape[2]]


def kernel(adj_DM, adj_D, adj_M, drg_embed, mic_embed, mix_embed,
           wd1, wd2, wm1, wm2, dm1, dm2, att):
    Nd = adj_D.shape[0]
    Nm = adj_M.shape[0]
    F = drg_embed.shape[1]
    H1 = wd1.shape[1]
    H2 = wd2.shape[1]

    BP = _ceil_to(Nd, _LANE)
    FP = _ceil_to(F, _LANE)
    H1P = _ceil_to(H1, _LANE)
    H2P = _ceil_to(H2, _LANE)

    def p2(x, r, c, dt=_BF):
        return jnp.pad(x, ((0, r - x.shape[0]), (0, c - x.shape[1]))).astype(dt)

    X = jnp.stack([p2(drg_embed, BP, FP), p2(mic_embed, BP, FP)])
    A = jnp.stack([p2(adj_D, BP, BP), p2(adj_M, BP, BP)])
    W1 = jnp.stack([p2(wd1, FP, H1P), p2(wm1, FP, H1P)])
    W2 = jnp.stack([p2(wd2, H1P, H2P), p2(wm2, H1P, H2P)])
    MIX = jnp.stack([p2(mix_embed[:Nd], BP, FP), p2(mix_embed[Nd:], BP, FP)])
    ATT = jnp.stack([p2(att[:Nd], BP, 1, _F32), p2(att[Nd:], BP, 1, _F32)])
    D1T = p2(dm1[:F], FP, H1P)
    D1B = p2(dm1[F:], H1P, H1P)
    D2T = p2(dm2[:H1], H1P, H2P)
    D2B = p2(dm2[H1:], H2P, H2P)
    AL = jnp.stack([p2(adj_DM[:Nd, :Nd], BP, BP), p2(adj_DM[Nd:, :Nd], BP, BP)])
    AR = jnp.stack([p2(adj_DM[:Nd, Nd:], BP, BP), p2(adj_DM[Nd:, Nd:], BP, BP)])

    def bspec(shape):
        return pl.BlockSpec((1,) + shape, lambda b: (b, 0, 0))

    def whole(shape):
        nd = len(shape)
        return pl.BlockSpec(shape, lambda b: (0,) * nd)

    params = pltpu.CompilerParams(dimension_semantics=("parallel",),
                                  vmem_limit_bytes=_VMEM)

    h1, h2, y1, h2s = pl.pallas_call(
        functools.partial(_branch_body, nrows=Nd),
        grid=(2,),
        in_specs=[bspec((BP, FP)), bspec((BP, BP)), bspec((FP, H1P)),
                  bspec((H1P, H2P)), bspec((BP, FP)), whole((FP, H1P)),
                  whole((H1P, H1P)), bspec((BP, 1))],
        out_specs=[bspec((Nd, H1)), bspec((Nd, H2)), bspec((BP, H1P)),
                   bspec((BP, H2P))],
        out_shape=[jax.ShapeDtypeStruct((2, Nd, H1), _F32),
                   jax.ShapeDtypeStruct((2, Nd, H2), _F32),
                   jax.ShapeDtypeStruct((2, BP, H1P), _BF),
                   jax.ShapeDtypeStruct((2, BP, H2P), _BF)],
        compiler_params=params,
    )(X, A, W1, W2, MIX, D1T, D1B, ATT)

    d1, y2 = pl.pallas_call(
        functools.partial(_hop1_body, nrows=Nd),
        grid=(2,),
        in_specs=[bspec((BP, BP)), bspec((BP, BP)), whole((2, BP, H1P)),
                  whole((H1P, H2P)), whole((H2P, H2P)), bspec((BP, H2P))],
        out_specs=[bspec((Nd, H1)), bspec((BP, H2P))],
        out_shape=[jax.ShapeDtypeStruct((2, Nd, H1), _F32),
                   jax.ShapeDtypeStruct((2, BP, H2P), _BF)],
        compiler_params=params,
    )(AL, AR, y1, D2T, D2B, h2s)

    d2 = pl.pallas_call(
        functools.partial(_hop2_body, nrows=Nd),
        grid=(2,),
        in_specs=[bspec((BP, BP)), bspec((BP, BP)), whole((2, BP, H2P))],
        out_specs=bspec((Nd, H2)),
        out_shape=jax.ShapeDtypeStruct((2, Nd, H2), _F32),
        compiler_params=params,
    )(AL, AR, y2)

    drg1hop = h1[0]
    drg2hop = h2[0]
    mic1hop = h1[1]
    mic2hop = h2[1]
    dm1hop = d1.reshape(Nd + Nm, H1)
    dm2hop = d2.reshape(Nd + Nm, H2)
    return drg1hop, drg2hop, mic1hop, mic2hop, dm1hop, dm2hop


# speedup vs baseline: 1.0081x; 1.0081x over previous
import functools

import jax
import jax.numpy as jnp
from jax.experimental import pallas as pl
from jax.experimental.pallas import tpu as pltpu

_NEG_SLOPE = 0.01
_LANE = 128
_VMEM = 64 * 1024 * 1024

_BF = jnp.bfloat16
_F32 = jnp.float32


def _lrelu(x):
    return jnp.where(x > 0, x, _NEG_SLOPE * x)


def _ceil_to(n, m):
    return ((n + m - 1) // m) * m


def _dot(a, b):
    return jnp.dot(a, b, preferred_element_type=jnp.float32)


def _branch_body(x_ref, a_ref, w1_ref, w2_ref, mix_ref, d1t_ref, d1b_ref,
                 att_ref, h1_ref, h2_ref, y1_ref, h2s_ref, *, nrows):
    a = a_ref[0]
    att = att_ref[0]

    t1 = _dot(x_ref[0], w1_ref[0]).astype(_BF)
    h1 = _lrelu(_dot(a, t1))
    h1_ref[0] = h1
    h1s = (att * h1).astype(_BF)

    t2 = _dot(h1.astype(_BF), w2_ref[0]).astype(_BF)
    h2 = _lrelu(_dot(a, t2))
    h2_ref[0] = h2
    h2s_ref[0] = (att * h2).astype(_BF)

    y1 = _dot(mix_ref[0], d1t_ref[...]) + _dot(h1s, d1b_ref[...])
    y1_ref[0] = y1.astype(_BF)


def _hop1_body(al_ref, ar_ref, y1_ref, d2t_ref, d2b_ref, h2s_ref,
               d1_ref, y2_ref, *, nrows):
    d1 = _lrelu(_dot(al_ref[0], y1_ref[0]) + _dot(ar_ref[0], y1_ref[1]))
    d1_ref[0] = d1[:nrows, :d1_ref.shape[2]]
    y2 = _dot(d1.astype(_BF), d2t_ref[...]) + _dot(h2s_ref[0], d2b_ref[...])
    y2_ref[0] = y2.astype(_BF)


def _hop2_body(al_ref, ar_ref, y2_ref, d2_ref, *, nrows):
    d2 = _lrelu(_dot(al_ref[0], y2_ref[0]) + _dot(ar_ref[0], y2_ref[1]))
    d2_ref[0] = d2[:nrows, :d2_ref.shape[2]]


def kernel(adj_DM, adj_D, adj_M, drg_embed, mic_embed, mix_embed,
           wd1, wd2, wm1, wm2, dm1, dm2, att):
    Nd = adj_D.shape[0]
    Nm = adj_M.shape[0]
    F = drg_embed.shape[1]
    H1 = wd1.shape[1]
    H2 = wd2.shape[1]

    BP = _ceil_to(Nd, _LANE)
    FP = _ceil_to(F, _LANE)
    H1P = _ceil_to(H1, _LANE)
    H2P = _ceil_to(H2, _LANE)

    def p2(x, r, c, dt=_BF):
        return jnp.pad(x, ((0, r - x.shape[0]), (0, c - x.shape[1]))).astype(dt)

    X = jnp.stack([p2(drg_embed, BP, FP), p2(mic_embed, BP, FP)])
    A = jnp.stack([p2(adj_D, BP, BP), p2(adj_M, BP, BP)])
    W1 = jnp.stack([p2(wd1, FP, H1P), p2(wm1, FP, H1P)])
    W2 = jnp.stack([p2(wd2, H1P, H2P), p2(wm2, H1P, H2P)])
    MIX = jnp.stack([p2(mix_embed[:Nd], BP, FP), p2(mix_embed[Nd:], BP, FP)])
    ATT = jnp.stack([p2(att[:Nd], BP, 1, _F32), p2(att[Nd:], BP, 1, _F32)])
    D1T = p2(dm1[:F], FP, H1P)
    D1B = p2(dm1[F:], H1P, H1P)
    D2T = p2(dm2[:H1], H1P, H2P)
    D2B = p2(dm2[H1:], H2P, H2P)
    AL = jnp.stack([p2(adj_DM[:Nd, :Nd], BP, BP), p2(adj_DM[Nd:, :Nd], BP, BP)])
    AR = jnp.stack([p2(adj_DM[:Nd, Nd:], BP, BP), p2(adj_DM[Nd:, Nd:], BP, BP)])

    def bspec(shape):
        return pl.BlockSpec((1,) + shape, lambda b: (b, 0, 0))

    def whole(shape):
        nd = len(shape)
        return pl.BlockSpec(shape, lambda b: (0,) * nd)

    params = pltpu.CompilerParams(dimension_semantics=("parallel",),
                                  vmem_limit_bytes=_VMEM)

    h1, h2, y1, h2s = pl.pallas_call(
        functools.partial(_branch_body, nrows=Nd),
        grid=(2,),
        in_specs=[bspec((BP, FP)), bspec((BP, BP)), bspec((FP, H1P)),
                  bspec((H1P, H2P)), bspec((BP, FP)), whole((FP, H1P)),
                  whole((H1P, H1P)), bspec((BP, 1))],
        out_specs=[bspec((BP, H1P)), bspec((BP, H2P)), bspec((BP, H1P)),
                   bspec((BP, H2P))],
        out_shape=[jax.ShapeDtypeStruct((2, BP, H1P), _F32),
                   jax.ShapeDtypeStruct((2, BP, H2P), _F32),
                   jax.ShapeDtypeStruct((2, BP, H1P), _BF),
                   jax.ShapeDtypeStruct((2, BP, H2P), _BF)],
        compiler_params=params,
    )(X, A, W1, W2, MIX, D1T, D1B, ATT)

    d1, y2 = pl.pallas_call(
        functools.partial(_hop1_body, nrows=Nd),
        grid=(2,),
        in_specs=[bspec((BP, BP)), bspec((BP, BP)), whole((2, BP, H1P)),
                  whole((H1P, H2P)), whole((H2P, H2P)), bspec((BP, H2P))],
        out_specs=[bspec((Nd, H1)), bspec((BP, H2P))],
        out_shape=[jax.ShapeDtypeStruct((2, Nd, H1), _F32),
                   jax.ShapeDtypeStruct((2, BP, H2P), _BF)],
        compiler_params=params,
    )(AL, AR, y1, D2T, D2B, h2s)

    d2 = pl.pallas_call(
        functools.partial(_hop2_body, nrows=Nd),
        grid=(2,),
        in_specs=[bspec((BP, BP)), bspec((BP, BP)), whole((2, BP, H2P))],
        out_specs=bspec((Nd, H2)),
        out_shape=jax.ShapeDtypeStruct((2, Nd, H2), _F32),
        compiler_params=params,
    )(AL, AR, y2)

    drg1hop = h1[0, :Nd, :H1]
    drg2hop = h2[0, :Nd, :H2]
    mic1hop = h1[1, :Nm, :H1]
    mic2hop = h2[1, :Nm, :H2]
    dm1hop = d1.reshape(Nd + Nm, H1)
    dm2hop = d2.reshape(Nd + Nm, H2)
    return drg1hop, drg2hop, mic1hop, mic2hop, dm1hop, dm2hop


# speedup vs baseline: 1.0710x; 1.0623x over previous
import jax
import jax.numpy as jnp
from jax.experimental import pallas as pl
from jax.experimental.pallas import tpu as pltpu

_NEG_SLOPE = 0.01
_VMEM = 64 * 1024 * 1024

_BF = jnp.bfloat16
_F32 = jnp.float32


def _lrelu(x):
    return jnp.where(x > 0, x, _NEG_SLOPE * x)


def _dot(a, b):
    return jnp.dot(a, b, preferred_element_type=jnp.float32)


def _branch_body(x_ref, a_ref, w1_ref, w2_ref, mix_ref, d1t_ref, d1b_ref,
                 att_ref, h1_ref, h2_ref, y1_ref, h2s_ref):
    a = a_ref[0]
    att = att_ref[0]

    t1 = _dot(x_ref[0], w1_ref[0]).astype(_BF)
    h1 = _lrelu(_dot(a, t1))
    h1_ref[0] = h1
    h1s = (att * h1).astype(_BF)

    t2 = _dot(h1.astype(_BF), w2_ref[0]).astype(_BF)
    h2 = _lrelu(_dot(a, t2))
    h2_ref[0] = h2
    h2s_ref[0] = (att * h2).astype(_BF)

    y1 = _dot(mix_ref[0], d1t_ref[...]) + _dot(h1s, d1b_ref[...])
    y1_ref[0] = y1.astype(_BF)


def _hop1_body(al_ref, ar_ref, y1_ref, d2t_ref, d2b_ref, h2s_ref,
               d1_ref, y2_ref):
    d1 = _lrelu(_dot(al_ref[0], y1_ref[0]) + _dot(ar_ref[0], y1_ref[1]))
    d1_ref[0] = d1
    y2 = _dot(d1.astype(_BF), d2t_ref[...]) + _dot(h2s_ref[0], d2b_ref[...])
    y2_ref[0] = y2.astype(_BF)


def _hop2_body(al_ref, ar_ref, y2_ref, d2_ref):
    d2_ref[0] = _lrelu(_dot(al_ref[0], y2_ref[0]) + _dot(ar_ref[0], y2_ref[1]))


def kernel(adj_DM, adj_D, adj_M, drg_embed, mic_embed, mix_embed,
           wd1, wd2, wm1, wm2, dm1, dm2, att):
    Nd = adj_D.shape[0]
    Nm = adj_M.shape[0]
    F = drg_embed.shape[1]
    H1 = wd1.shape[1]
    H2 = wd2.shape[1]
    N = Nd + Nm

    bf = lambda x: x.astype(_BF)

    X = jnp.stack([bf(drg_embed), bf(mic_embed)])
    A = jnp.stack([bf(adj_D), bf(adj_M)])
    W1 = jnp.stack([bf(wd1), bf(wm1)])
    W2 = jnp.stack([bf(wd2), bf(wm2)])
    MIX = bf(mix_embed).reshape(2, Nd, F)
    ATT = att.reshape(2, Nd, 1)
    D1T = bf(dm1[:F])
    D1B = bf(dm1[F:])
    D2T = bf(dm2[:H1])
    D2B = bf(dm2[H1:])
    AL = jnp.stack([bf(adj_DM[:Nd, :Nd]), bf(adj_DM[Nd:, :Nd])])
    AR = jnp.stack([bf(adj_DM[:Nd, Nd:]), bf(adj_DM[Nd:, Nd:])])

    def bspec(shape):
        return pl.BlockSpec((1,) + shape, lambda b: (b, 0, 0))

    def whole(shape):
        nd = len(shape)
        return pl.BlockSpec(shape, lambda b: (0,) * nd)

    params = pltpu.CompilerParams(dimension_semantics=("arbitrary",),
                                  vmem_limit_bytes=_VMEM)

    h1, h2, y1, h2s = pl.pallas_call(
        _branch_body,
        grid=(2,),
        in_specs=[bspec((Nd, F)), bspec((Nd, Nd)), bspec((F, H1)),
                  bspec((H1, H2)), bspec((Nd, F)), whole((F, H1)),
                  whole((H1, H1)), bspec((Nd, 1))],
        out_specs=[bspec((Nd, H1)), bspec((Nd, H2)), bspec((Nd, H1)),
                   bspec((Nd, H2))],
        out_shape=[jax.ShapeDtypeStruct((2, Nd, H1), _F32),
                   jax.ShapeDtypeStruct((2, Nd, H2), _F32),
                   jax.ShapeDtypeStruct((2, Nd, H1), _BF),
                   jax.ShapeDtypeStruct((2, Nd, H2), _BF)],
        compiler_params=params,
    )(X, A, W1, W2, MIX, D1T, D1B, ATT)

    d1, y2 = pl.pallas_call(
        _hop1_body,
        grid=(2,),
        in_specs=[bspec((Nd, Nd)), bspec((Nd, Nm)), whole((2, Nd, H1)),
                  whole((H1, H2)), whole((H2, H2)), bspec((Nd, H2))],
        out_specs=[bspec((Nd, H1)), bspec((Nd, H2))],
        out_shape=[jax.ShapeDtypeStruct((2, Nd, H1), _F32),
                   jax.ShapeDtypeStruct((2, Nd, H2), _BF)],
        compiler_params=params,
    )(AL, AR, y1, D2T, D2B, h2s)

    d2 = pl.pallas_call(
        _hop2_body,
        grid=(2,),
        in_specs=[bspec((Nd, Nd)), bspec((Nd, Nm)), whole((2, Nd, H2))],
        out_specs=bspec((Nd, H2)),
        out_shape=jax.ShapeDtypeStruct((2, Nd, H2), _F32),
        compiler_params=params,
    )(AL, AR, y2)

    return h1[0], h2[0], h1[1], h2[1], d1.reshape(N, H1), d2.reshape(N, H2)
